# Initial kernel scaffold; baseline (speedup 1.0000x reference)
import jax
import jax.numpy as jnp
from jax.experimental import pallas as pl


def kernel(idx, emb, ln1_g, ln1_b, ln2_g, ln2_b, Wr, Wk, Wv, Wo, conf_w, crit_w, aff_w, bridge_W1, bridge_W2, ffn_W1, ffn_W2, trans_W1, trans_W2, lnout_g, lnout_b, head_W, capital_shares):
    raise NotImplementedError("write your pallas kernel here")


# speedup vs baseline: 16.5840x; 16.5840x over previous
import functools

import jax
import jax.numpy as jnp
from jax import lax
from jax.experimental import pallas as pl
from jax.experimental.pallas import tpu as pltpu
from jax.experimental.pallas import tpu_sc as plsc

F32 = jnp.float32
BF16 = jnp.bfloat16
DECAY = 0.95
LN_EPS = 1e-5
Q_CHUNK = 256


HI = jax.lax.Precision.HIGHEST


def _dot_hi(a, b):
    return jnp.dot(a, b, preferred_element_type=F32)


def _embed_gather(emb, idx_flat):
    V, C = emb.shape
    B = idx_flat.shape[0]
    info = plsc.get_sparse_core_info()
    nw = info.num_cores * info.num_subcores
    b_per_w = B // nw
    mesh = plsc.VectorSubcoreMesh(core_axis_name="c", subcore_axis_name="s")

    @functools.partial(
        pl.kernel,
        mesh=mesh,
        out_type=jax.ShapeDtypeStruct((B, C), F32),
        scratch_types=[
            pltpu.VMEM((b_per_w,), jnp.int32),
            pltpu.VMEM((b_per_w, C), F32),
            pltpu.SemaphoreType.DMA,
        ],
    )
    def gk(table_hbm, idx_hbm, out_hbm, idx_v, rows_v, sem):
        wid = lax.axis_index("s") * info.num_cores + lax.axis_index("c")
        base = wid * b_per_w
        pltpu.sync_copy(idx_hbm.at[pl.ds(base, b_per_w)], idx_v)
        pltpu.async_copy(table_hbm.at[idx_v], rows_v, sem).wait()
        pltpu.sync_copy(rows_v, out_hbm.at[pl.ds(base, b_per_w)])

    return gk(emb, idx_flat)


def _qkv(x, wr, wk, wv, g, b, v0):
    T, C = x.shape
    TT = 512
    mix = v0 is not None

    def body(*refs):
        if mix:
            x_ref, wr_ref, wk_ref, wv_ref, g_ref, b_ref, v0_ref, r_o, k_o, v_o = refs
        else:
            x_ref, wr_ref, wk_ref, wv_ref, g_ref, b_ref, r_o, k_o, v_o = refs
        xv = x_ref[...]
        m = jnp.mean(xv, axis=1, keepdims=True)
        var = jnp.mean((xv - m) ** 2, axis=1, keepdims=True)
        xl = (xv - m) / jnp.sqrt(var + LN_EPS) * g_ref[...] + b_ref[...]
        r_o[...] = _dot_hi(xl, wr_ref[...])
        k_o[...] = _dot_hi(xl, wk_ref[...])
        vv = _dot_hi(xl, wv_ref[...])
        if mix:
            vv = 0.5 * (vv + v0_ref[...])
        v_o[...] = vv

    in_specs = [
        pl.BlockSpec((TT, C), lambda i: (i, 0)),
        pl.BlockSpec((C, C), lambda i: (0, 0)),
        pl.BlockSpec((C, C), lambda i: (0, 0)),
        pl.BlockSpec((C, C), lambda i: (0, 0)),
        pl.BlockSpec((1, C), lambda i: (0, 0)),
        pl.BlockSpec((1, C), lambda i: (0, 0)),
    ]
    args = [x, wr, wk, wv, g.reshape(1, C), b.reshape(1, C)]
    if mix:
        in_specs.append(pl.BlockSpec((TT, C), lambda i: (i, 0)))
        args.append(v0)
    return pl.pallas_call(
        body,
        grid=(T // TT,),
        in_specs=in_specs,
        out_specs=[pl.BlockSpec((TT, C), lambda i: (i, 0))] * 3,
        out_shape=[jax.ShapeDtypeStruct((T, C), F32)] * 3,
    )(*args)


def _time_mix(r, k, v, H, D):
    T = r.shape[0]
    Q = Q_CHUNK
    CN = T // Q
    r3 = r.reshape(T, H, D).transpose(1, 0, 2)
    k3 = k.reshape(T, H, D).transpose(1, 0, 2)
    v3 = v.reshape(T, H, D).transpose(1, 0, 2)

    ar = jnp.arange(Q, dtype=F32)
    diff = ar[:, None] - ar[None, :]
    M = jnp.where(diff >= 0, DECAY ** jnp.maximum(diff, 0.0), 0.0).astype(F32)
    dr = (DECAY ** (ar + 1.0)).reshape(Q, 1).astype(F32)
    dk = (DECAY ** (Q - 1.0 - ar)).reshape(Q, 1).astype(F32)
    decay_q = float(DECAY) ** Q

    def body(r_ref, k_ref, v_ref, m_ref, dr_ref, dk_ref, o_ref, s_ref):
        c = pl.program_id(0)

        @pl.when(c == 0)
        def _():
            s_ref[...] = jnp.zeros_like(s_ref)

        mv = m_ref[...]
        drv = dr_ref[...]
        dkv = dk_ref[...]
        for h in range(H):
            rh = r_ref[h].astype(BF16).astype(F32)
            kh = k_ref[h]
            vh = v_ref[h]
            sh = s_ref[h]
            p = lax.dot_general(rh, kh, (((1,), (1,)), ((), ())),
                                preferred_element_type=F32, precision=HI)
            o = (jnp.dot(p * mv, vh, preferred_element_type=F32, precision=HI)
                 + jnp.dot(rh * drv, sh, preferred_element_type=F32,
                           precision=HI))
            o_ref[h] = o
            s_ref[h] = decay_q * sh + lax.dot_general(
                kh * dkv, vh, (((0,), (0,)), ((), ())),
                preferred_element_type=F32, precision=HI)

    out = pl.pallas_call(
        body,
        grid=(CN,),
        in_specs=[
            pl.BlockSpec((H, Q, D), lambda c: (0, c, 0)),
            pl.BlockSpec((H, Q, D), lambda c: (0, c, 0)),
            pl.BlockSpec((H, Q, D), lambda c: (0, c, 0)),
            pl.BlockSpec((Q, Q), lambda c: (0, 0)),
            pl.BlockSpec((Q, 1), lambda c: (0, 0)),
            pl.BlockSpec((Q, 1), lambda c: (0, 0)),
        ],
        out_specs=pl.BlockSpec((H, Q, D), lambda c: (0, c, 0)),
        out_shape=jax.ShapeDtypeStruct((H, T, D), F32),
        scratch_shapes=[pltpu.VMEM((H, D, D), F32)],
    )(r3, k3, v3, M, dr, dk)
    return out.transpose(1, 0, 2).reshape(T, H * D)


def _post(out_tm, x, wo, g2, b2, conf_wt, crit_col, aff, cap_row, w1h, w1r, w2b):
    T, C = x.shape
    E = conf_wt.shape[1]
    R = w1h.shape[1]
    TT = 512

    def body(o_ref, x_ref, wo_ref, g_ref, b_ref, cw_ref, cr_ref, aw_ref,
             cap_ref, w1h_ref, w1r_ref, w2_ref, xn_ref, h_ref, t_ref, wm_ref):
        o = o_ref[...]
        xn = x_ref[...] + _dot_hi(o, wo_ref[...])
        xn_ref[...] = xn
        m = jnp.mean(xn, axis=1, keepdims=True)
        var = jnp.mean((xn - m) ** 2, axis=1, keepdims=True)
        h = (xn - m) / jnp.sqrt(var + LN_EPS) * g_ref[...] + b_ref[...]
        h_ref[...] = h
        conf = jax.nn.sigmoid(_dot_hi(h, cw_ref[...]))
        dif = jax.nn.sigmoid(_dot_hi(h, cr_ref[...]))
        affv = _dot_hi(h, aw_ref[...])
        bids = conf * cap_ref[...] * dif + 0.1 * jnp.tanh(affv)
        eio = lax.broadcasted_iota(jnp.int32, (TT, E), 1)
        m1 = jnp.max(bids, axis=1, keepdims=True)
        i1 = jnp.min(jnp.where(bids == m1, eio, E), axis=1, keepdims=True)
        rest = jnp.where(eio == i1, -1e30, bids)
        m2 = jnp.max(rest, axis=1, keepdims=True)
        i2 = jnp.min(jnp.where(rest == m2, eio, E), axis=1, keepdims=True)
        w1 = jax.nn.sigmoid(m1 - m2)
        wm_ref[...] = (jnp.where(eio == i1, w1, 0.0)
                       + jnp.where(eio == i2, 1.0 - w1, 0.0))
        pre = jax.nn.relu(_dot_hi(h, w1h_ref[...]) + _dot_hi(o, w1r_ref[...]))
        t_ref[...] = h + _dot_hi(pre, w2_ref[...])

    return pl.pallas_call(
        body,
        grid=(T // TT,),
        in_specs=[
            pl.BlockSpec((TT, C), lambda i: (i, 0)),
            pl.BlockSpec((TT, C), lambda i: (i, 0)),
            pl.BlockSpec((C, C), lambda i: (0, 0)),
            pl.BlockSpec((1, C), lambda i: (0, 0)),
            pl.BlockSpec((1, C), lambda i: (0, 0)),
            pl.BlockSpec((C, E), lambda i: (0, 0)),
            pl.BlockSpec((C, 1), lambda i: (0, 0)),
            pl.BlockSpec((C, E), lambda i: (0, 0)),
            pl.BlockSpec((1, E), lambda i: (0, 0)),
            pl.BlockSpec((C, R), lambda i: (0, 0)),
            pl.BlockSpec((C, R), lambda i: (0, 0)),
            pl.BlockSpec((R, C), lambda i: (0, 0)),
        ],
        out_specs=[
            pl.BlockSpec((TT, C), lambda i: (i, 0)),
            pl.BlockSpec((TT, C), lambda i: (i, 0)),
            pl.BlockSpec((TT, C), lambda i: (i, 0)),
            pl.BlockSpec((TT, E), lambda i: (i, 0)),
        ],
        out_shape=[
            jax.ShapeDtypeStruct((T, C), F32),
            jax.ShapeDtypeStruct((T, C), F32),
            jax.ShapeDtypeStruct((T, C), F32),
            jax.ShapeDtypeStruct((T, E), F32),
        ],
    )(out_tm, x, wo, g2.reshape(1, C), b2.reshape(1, C), conf_wt, crit_col,
      aff, cap_row, w1h, w1r, w2b)


def _moe(xn, h, t_in, wmat, w1_all, w2_all, e_r, final_ln):
    T, C = h.shape
    E = w1_all.shape[0]
    TT = 1024

    def body(*refs):
        if final_ln is not None:
            (h_ref, t_ref, xn_ref, wm_ref, w1_ref, w2_ref, g_ref, b_ref,
             o_ref) = refs
        else:
            h_ref, t_ref, xn_ref, wm_ref, w1_ref, w2_ref, o_ref = refs
        e = pl.program_id(1)

        @pl.when(e == 0)
        def _():
            o_ref[...] = xn_ref[...]

        is_ffn = e < e_r
        inp = jnp.where(is_ffn, h_ref[...], t_ref[...])
        z = _dot_hi(inp, w1_ref[0])
        act = jnp.where(is_ffn, jnp.square(jnp.maximum(z, 0.0)), jax.nn.gelu(z))
        oe = _dot_hi(act, w2_ref[0])
        eio = lax.broadcasted_iota(jnp.int32, (TT, E), 1)
        wtok = jnp.sum(jnp.where(eio == e, wm_ref[...], 0.0), axis=1,
                       keepdims=True)
        o_ref[...] += oe * wtok

        if final_ln is not None:
            @pl.when(e == E - 1)
            def _():
                xo = o_ref[...]
                m = jnp.mean(xo, axis=1, keepdims=True)
                var = jnp.mean((xo - m) ** 2, axis=1, keepdims=True)
                o_ref[...] = ((xo - m) / jnp.sqrt(var + LN_EPS) * g_ref[...]
                              + b_ref[...])

    in_specs = [
        pl.BlockSpec((TT, C), lambda i, e: (i, 0)),
        pl.BlockSpec((TT, C), lambda i, e: (i, 0)),
        pl.BlockSpec((TT, C), lambda i, e: (i, 0)),
        pl.BlockSpec((TT, E), lambda i, e: (i, 0)),
        pl.BlockSpec((1, C, C), lambda i, e: (e, 0, 0)),
        pl.BlockSpec((1, C, C), lambda i, e: (e, 0, 0)),
    ]
    args = [h, t_in, xn, wmat, w1_all, w2_all]
    if final_ln is not None:
        g, b = final_ln
        in_specs.append(pl.BlockSpec((1, C), lambda i, e: (0, 0)))
        in_specs.append(pl.BlockSpec((1, C), lambda i, e: (0, 0)))
        args.append(g.reshape(1, C))
        args.append(b.reshape(1, C))
    return pl.pallas_call(
        body,
        grid=(T // TT, E),
        in_specs=in_specs,
        out_specs=pl.BlockSpec((TT, C), lambda i, e: (i, 0)),
        out_shape=jax.ShapeDtypeStruct((T, C), F32),
    )(*args)


def _head(xf, head_w):
    T, C = xf.shape
    V = head_w.shape[1]
    VT = 1024

    def body(x_ref, w_ref, o_ref):
        o_ref[...] = _dot_hi(x_ref[...], w_ref[...])

    return pl.pallas_call(
        body,
        grid=(V // VT,),
        in_specs=[
            pl.BlockSpec((T, C), lambda j: (0, 0)),
            pl.BlockSpec((C, VT), lambda j: (0, j)),
        ],
        out_specs=pl.BlockSpec((T, VT), lambda j: (0, j)),
        out_shape=jax.ShapeDtypeStruct((T, V), F32),
    )(xf, head_w)


def kernel(idx, emb, ln1_g, ln1_b, ln2_g, ln2_b, Wr, Wk, Wv, Wo, conf_w,
           crit_w, aff_w, bridge_W1, bridge_W2, ffn_W1, ffn_W2, trans_W1,
           trans_W2, lnout_g, lnout_b, head_W, capital_shares):
    B, T = idx.shape
    V, C = emb.shape
    L = Wr.shape[0]
    H, D = 12, C // 12
    e_r = ffn_W1.shape[1]
    e_t = trans_W1.shape[1]
    E = e_r + e_t

    x = _embed_gather(emb, idx.reshape(-1).astype(jnp.int32))
    w1h = bridge_W1[:C]
    w1r = bridge_W1[C:]

    v0 = None
    for l in range(L):
        r, k, v = _qkv(x, Wr[l], Wk[l], Wv[l], ln1_g[l], ln1_b[l], v0)
        if l == 0:
            v0 = v
        tm = _time_mix(r, k, v, H, D)
        xn, h, t_in, wmat = _post(
            tm, x, Wo[l], ln2_g[l], ln2_b[l], conf_w[l].T,
            crit_w[l].reshape(C, 1), aff_w[l],
            capital_shares[l].reshape(1, E), w1h, w1r, bridge_W2)
        w1_all = jnp.concatenate([ffn_W1[l], trans_W1[l]], axis=0)
        w2_all = jnp.concatenate([ffn_W2[l], trans_W2[l]], axis=0)
        fin = (lnout_g, lnout_b) if l == L - 1 else None
        x = _moe(xn, h, t_in, wmat, w1_all, w2_all, e_r, fin)

    logits = _head(x, head_W)
    return logits.reshape(B, T, V)
